# baseline (speedup 1.0000x reference)
import functools

import jax
import jax.numpy as jnp
from jax import lax
from jax.experimental import pallas as pl
from jax.experimental.pallas import tpu as pltpu
from jax.experimental.pallas import tpu_sc as plsc

N = 10000
D = 128
E = 320000

NC = 2
NS = 16
NW = NC * NS
NPAD = 10240
ROWS_PER_TILE = NPAD // NS
EW = 10240
EPAD = NW * EW
K = 80
CHUNKS = EW // K
NBUF = 4
PADCH = NBUF
DEGW = 128

_BLK = 1000
_GRID = N // _BLK

_sc_mesh = plsc.VectorSubcoreMesh(
    core_axis_name="c", subcore_axis_name="s", num_cores=NC, num_subcores=NS
)


@functools.partial(
    pl.kernel,
    out_type=jax.ShapeDtypeStruct((NC, NPAD, DEGW), jnp.float32),
    mesh=_sc_mesh,
    scratch_types=[
        pltpu.VMEM_SHARED((NPAD, DEGW), jnp.float32),
        pltpu.VMEM((K,), jnp.int32),
        pltpu.VMEM((K, DEGW), jnp.float32),
    ],
)
def _deg_kernel(dst_hbm, ones_hbm, zeros_hbm, out_hbm, deg_sh, idx_v, ones_v):
    c = lax.axis_index("c")
    s = lax.axis_index("s")
    base = (s * NC + c) * EW
    soff = s * ROWS_PER_TILE
    pltpu.sync_copy(zeros_hbm, deg_sh.at[pl.ds(soff, ROWS_PER_TILE)])
    pltpu.sync_copy(ones_hbm, ones_v)
    plsc.subcore_barrier()

    def body(j, carry):
        pltpu.sync_copy(dst_hbm.at[pl.ds(base + j * K, K)], idx_v)
        pltpu.sync_copy(ones_v, deg_sh.at[idx_v], add=True)
        return carry

    lax.fori_loop(0, CHUNKS, body, 0)
    plsc.subcore_barrier()
    pltpu.sync_copy(
        deg_sh.at[pl.ds(soff, ROWS_PER_TILE)],
        out_hbm.at[c, pl.ds(soff, ROWS_PER_TILE)],
    )


@functools.partial(
    pl.kernel,
    out_type=jax.ShapeDtypeStruct((NC, NPAD, D), jnp.float32),
    mesh=_sc_mesh,
    scratch_types=[
        pltpu.VMEM_SHARED((NPAD, D), jnp.float32),
        [pltpu.VMEM((2, K), jnp.int32) for _ in range(NBUF)],
        [pltpu.VMEM((K, D), jnp.float32) for _ in range(NBUF)],
        [pltpu.SemaphoreType.DMA for _ in range(NBUF)],
        [pltpu.SemaphoreType.DMA for _ in range(NBUF)],
    ],
)
def _agg_kernel(hp_hbm, idx_hbm, zeros_hbm, out_hbm,
                acc_sh, idxb, rowsb, isem, gsem):
    c = lax.axis_index("c")
    s = lax.axis_index("s")
    wid = s * NC + c
    soff = s * ROWS_PER_TILE
    pltpu.sync_copy(zeros_hbm, acc_sh.at[pl.ds(soff, ROWS_PER_TILE)])

    for b in range(NBUF):
        pltpu.async_copy(idx_hbm.at[wid, b], idxb[b], isem[b])
    for b in range(2):
        pltpu.make_async_copy(idx_hbm.at[wid, b], idxb[b], isem[b]).wait()
        pltpu.async_copy(hp_hbm.at[idxb[b].at[0]], rowsb[b], gsem[b])
    plsc.subcore_barrier()

    def quad(p, carry):
        j0 = p * NBUF
        for i in range(NBUF):
            j = j0 + i
            cur = i
            nx2 = (i + 2) % NBUF
            pltpu.make_async_copy(idx_hbm.at[wid, j + 2], idxb[nx2],
                                  isem[nx2]).wait()
            pltpu.async_copy(hp_hbm.at[idxb[nx2].at[0]], rowsb[nx2],
                             gsem[nx2])
            pltpu.make_async_copy(hp_hbm.at[idxb[cur].at[0]], rowsb[cur],
                                  gsem[cur]).wait()
            pltpu.sync_copy(rowsb[cur], acc_sh.at[idxb[cur].at[1]], add=True)
            pltpu.async_copy(idx_hbm.at[wid, j + NBUF], idxb[cur], isem[cur])
        return carry

    lax.fori_loop(0, CHUNKS // NBUF, quad, 0)

    for b in range(2):
        pltpu.make_async_copy(hp_hbm.at[idxb[b].at[0]], rowsb[b],
                              gsem[b]).wait()
    for b in range(2, NBUF):
        pltpu.make_async_copy(idx_hbm.at[wid, b], idxb[b], isem[b]).wait()

    plsc.subcore_barrier()
    pltpu.sync_copy(
        acc_sh.at[pl.ds(soff, ROWS_PER_TILE)],
        out_hbm.at[c, pl.ds(soff, ROWS_PER_TILE)],
    )


def _dot(a, b):
    return lax.dot_general(
        a, b, (((1,), (0,)), ((), ())),
        precision=lax.Precision.HIGHEST,
        preferred_element_type=jnp.float32,
    )


def _tc_first_body(degp_ref, x_ref, w_ref, h_ref, dinv_ref):
    dp = degp_ref[...]
    deg = dp[0, :, 0:1] + dp[1, :, 0:1] + 1.0
    dinv = lax.rsqrt(deg)
    dinv_ref[...] = dinv
    h_ref[...] = _dot(x_ref[...] * dinv, w_ref[...])


def _tc_mid_body(agg_ref, hp_ref, dinv_ref, b_ref, w_ref, out_ref):
    a = agg_ref[...]
    dinv = dinv_ref[...]
    o = (a[0] + a[1] + hp_ref[...]) * dinv + b_ref[...]
    out_ref[...] = _dot(jnp.maximum(o, 0.0) * dinv, w_ref[...])


def _tc_last_body(agg_ref, hp_ref, dinv_ref, b_ref, out_ref):
    a = agg_ref[...]
    out_ref[...] = (a[0] + a[1] + hp_ref[...]) * dinv_ref[...] + b_ref[...]


_spec_agg = pl.BlockSpec((2, _BLK, D), lambda i: (0, i, 0))
_spec_deg = pl.BlockSpec((2, _BLK, DEGW), lambda i: (0, i, 0))
_spec_row = pl.BlockSpec((_BLK, D), lambda i: (i, 0))
_spec_col = pl.BlockSpec((_BLK, 1), lambda i: (i, 0))
_spec_b = pl.BlockSpec((1, D), lambda i: (0, 0))
_spec_w = pl.BlockSpec((D, D), lambda i: (0, 0))

_tc_first = pl.pallas_call(
    _tc_first_body,
    grid=(_GRID,),
    in_specs=[_spec_deg, _spec_row, _spec_w],
    out_specs=[_spec_row, _spec_col],
    out_shape=[
        jax.ShapeDtypeStruct((N, D), jnp.float32),
        jax.ShapeDtypeStruct((N, 1), jnp.float32),
    ],
)

_tc_mid = pl.pallas_call(
    _tc_mid_body,
    grid=(_GRID,),
    in_specs=[_spec_agg, _spec_row, _spec_col, _spec_b, _spec_w],
    out_specs=_spec_row,
    out_shape=jax.ShapeDtypeStruct((N, D), jnp.float32),
)

_tc_last = pl.pallas_call(
    _tc_last_body,
    grid=(_GRID,),
    in_specs=[_spec_agg, _spec_row, _spec_col, _spec_b],
    out_specs=_spec_row,
    out_shape=jax.ShapeDtypeStruct((N, D), jnp.float32),
)


def kernel(x, edge_index, W0, b0, W1, b1, W2, b2, W3, b3, W4, b4):
    src = edge_index[0]
    dst = edge_index[1]
    pad = EPAD - E
    srcp = jnp.concatenate([src, jnp.zeros((pad,), jnp.int32)])
    dstp = jnp.concatenate([dst, jnp.full((pad,), N, jnp.int32)])
    idx = jnp.stack(
        [srcp.reshape(NW, CHUNKS, K), dstp.reshape(NW, CHUNKS, K)], axis=2
    )
    padi = jnp.stack(
        [jnp.zeros((NW, PADCH, K), jnp.int32),
         jnp.full((NW, PADCH, K), N, jnp.int32)], axis=2
    )
    idxall = jnp.concatenate([idx, padi], axis=1)

    zeros_deg = jnp.zeros((ROWS_PER_TILE, DEGW), jnp.float32)
    ones_deg = jnp.ones((K, DEGW), jnp.float32)
    zeros_acc = jnp.zeros((ROWS_PER_TILE, D), jnp.float32)

    degp = _deg_kernel(dstp, ones_deg, zeros_deg)
    h, dinv = _tc_first(degp, x, W0)

    bs = [b0, b1, b2, b3]
    Ws = [W1, W2, W3, W4]
    for i in range(4):
        agg = _agg_kernel(h, idxall, zeros_acc)
        h = _tc_mid(agg, h, dinv, bs[i].reshape(1, D), Ws[i])
    agg = _agg_kernel(h, idxall, zeros_acc)
    return _tc_last(agg, h, dinv, b4.reshape(1, D))


# speedup vs baseline: 1.0098x; 1.0098x over previous
import functools

import jax
import jax.numpy as jnp
from jax import lax
from jax.experimental import pallas as pl
from jax.experimental.pallas import tpu as pltpu
from jax.experimental.pallas import tpu_sc as plsc

N = 10000
D = 128
E = 320000

NC = 2
NS = 16
NW = NC * NS
NPAD = 10240
ROWS_PER_TILE = NPAD // NS
EW = 10240
EPAD = NW * EW
K = 80
CHUNKS = EW // K
NBUF = 4
PADCH = NBUF
DEGW = 128

_BLK = 1000
_GRID = N // _BLK

_sc_mesh = plsc.VectorSubcoreMesh(
    core_axis_name="c", subcore_axis_name="s", num_cores=NC, num_subcores=NS
)


@functools.partial(
    pl.kernel,
    out_type=jax.ShapeDtypeStruct((NC, NPAD, DEGW), jnp.float32),
    mesh=_sc_mesh,
    scratch_types=[
        pltpu.VMEM_SHARED((NPAD, DEGW), jnp.float32),
        pltpu.VMEM((K,), jnp.int32),
        pltpu.VMEM((K, DEGW), jnp.float32),
    ],
)
def _deg_kernel(dst_hbm, ones_hbm, zeros_hbm, out_hbm, deg_sh, idx_v, ones_v):
    c = lax.axis_index("c")
    s = lax.axis_index("s")
    base = (s * NC + c) * EW
    soff = s * ROWS_PER_TILE
    pltpu.sync_copy(zeros_hbm, deg_sh.at[pl.ds(soff, ROWS_PER_TILE)])
    pltpu.sync_copy(ones_hbm, ones_v)
    plsc.subcore_barrier()

    def body(j, carry):
        pltpu.sync_copy(dst_hbm.at[pl.ds(base + j * K, K)], idx_v)
        pltpu.sync_copy(ones_v, deg_sh.at[idx_v], add=True)
        return carry

    lax.fori_loop(0, CHUNKS, body, 0)
    plsc.subcore_barrier()
    pltpu.sync_copy(
        deg_sh.at[pl.ds(soff, ROWS_PER_TILE)],
        out_hbm.at[c, pl.ds(soff, ROWS_PER_TILE)],
    )


@functools.partial(
    pl.kernel,
    out_type=jax.ShapeDtypeStruct((NC, NPAD, D), jnp.float32),
    mesh=_sc_mesh,
    scratch_types=[
        pltpu.VMEM_SHARED((NPAD, D), jnp.float32),
        [pltpu.VMEM((2, K), jnp.int32) for _ in range(NBUF)],
        [pltpu.VMEM((K, D), jnp.float32) for _ in range(NBUF)],
        [pltpu.SemaphoreType.DMA for _ in range(NBUF)],
        [pltpu.SemaphoreType.DMA for _ in range(NBUF)],
    ],
)
def _agg_kernel(hp_hbm, idx_hbm, zeros_hbm, out_hbm,
                acc_sh, idxb, rowsb, isem, gsem):
    c = lax.axis_index("c")
    s = lax.axis_index("s")
    wid = s * NC + c
    soff = s * ROWS_PER_TILE
    pltpu.sync_copy(zeros_hbm, acc_sh.at[pl.ds(soff, ROWS_PER_TILE)])

    for b in range(NBUF):
        pltpu.async_copy(idx_hbm.at[wid, b], idxb[b], isem[b])
    for b in range(2):
        pltpu.make_async_copy(idx_hbm.at[wid, b], idxb[b], isem[b]).wait()
        pltpu.async_copy(hp_hbm.at[idxb[b].at[0]], rowsb[b], gsem[b])
    plsc.subcore_barrier()

    def quad(p, carry):
        j0 = p * NBUF
        for i in range(NBUF):
            j = j0 + i
            cur = i
            nx2 = (i + 2) % NBUF
            pltpu.make_async_copy(idx_hbm.at[wid, j + 2], idxb[nx2],
                                  isem[nx2]).wait()
            pltpu.async_copy(hp_hbm.at[idxb[nx2].at[0]], rowsb[nx2],
                             gsem[nx2])
            pltpu.make_async_copy(hp_hbm.at[idxb[cur].at[0]], rowsb[cur],
                                  gsem[cur]).wait()
            pltpu.sync_copy(rowsb[cur], acc_sh.at[idxb[cur].at[1]], add=True)
            pltpu.async_copy(idx_hbm.at[wid, j + NBUF], idxb[cur], isem[cur])
        return carry

    lax.fori_loop(0, CHUNKS // NBUF, quad, 0)

    for b in range(2):
        pltpu.make_async_copy(hp_hbm.at[idxb[b].at[0]], rowsb[b],
                              gsem[b]).wait()
    for b in range(2, NBUF):
        pltpu.make_async_copy(idx_hbm.at[wid, b], idxb[b], isem[b]).wait()

    plsc.subcore_barrier()
    pltpu.sync_copy(
        acc_sh.at[pl.ds(soff, ROWS_PER_TILE)],
        out_hbm.at[c, pl.ds(soff, ROWS_PER_TILE)],
    )


def _dot(a, b):
    return lax.dot_general(
        a, b, (((1,), (0,)), ((), ())),
        precision=lax.Precision.HIGHEST,
        preferred_element_type=jnp.float32,
    )


def _tc_first_body(degp_ref, x_ref, w_ref, h_ref, dinv_ref):
    dp = degp_ref[...]
    deg = dp[0, :, 0:1] + dp[1, :, 0:1] + 1.0
    dinv = lax.rsqrt(deg)
    dinv_ref[...] = dinv
    h_ref[...] = _dot(x_ref[...] * dinv, w_ref[...])


def _tc_mid_body(agg_ref, hp_ref, dinv_ref, b_ref, w_ref, out_ref):
    a = agg_ref[...]
    dinv = dinv_ref[...]
    o = (a[0] + a[1] + hp_ref[...]) * dinv + b_ref[...]
    out_ref[...] = _dot(jnp.maximum(o, 0.0) * dinv, w_ref[...])


def _tc_last_body(agg_ref, hp_ref, dinv_ref, b_ref, out_ref):
    a = agg_ref[...]
    out_ref[...] = (a[0] + a[1] + hp_ref[...]) * dinv_ref[...] + b_ref[...]


_spec_agg = pl.BlockSpec((2, _BLK, D), lambda i: (0, i, 0))
_spec_deg = pl.BlockSpec((2, _BLK, DEGW), lambda i: (0, i, 0))
_spec_row = pl.BlockSpec((_BLK, D), lambda i: (i, 0))
_spec_col = pl.BlockSpec((_BLK, 1), lambda i: (i, 0))
_spec_b = pl.BlockSpec((1, D), lambda i: (0, 0))
_spec_w = pl.BlockSpec((D, D), lambda i: (0, 0))

_tc_first = pl.pallas_call(
    _tc_first_body,
    grid=(_GRID,),
    in_specs=[_spec_deg, _spec_row, _spec_w],
    out_specs=[_spec_row, _spec_col],
    out_shape=[
        jax.ShapeDtypeStruct((N, D), jnp.float32),
        jax.ShapeDtypeStruct((N, 1), jnp.float32),
    ],
)

_tc_mid = pl.pallas_call(
    _tc_mid_body,
    grid=(_GRID,),
    in_specs=[_spec_agg, _spec_row, _spec_col, _spec_b, _spec_w],
    out_specs=_spec_row,
    out_shape=jax.ShapeDtypeStruct((N, D), jnp.float32),
)

_tc_last = pl.pallas_call(
    _tc_last_body,
    grid=(_GRID,),
    in_specs=[_spec_agg, _spec_row, _spec_col, _spec_b],
    out_specs=_spec_row,
    out_shape=jax.ShapeDtypeStruct((N, D), jnp.float32),
)


def kernel(x, edge_index, W0, b0, W1, b1, W2, b2, W3, b3, W4, b4):
    src = edge_index[0]
    dst = edge_index[1]
    pad = EPAD - E
    spare = NPAD - N
    pad_dst = N + (jnp.arange(pad, dtype=jnp.int32) % spare)
    srcp = jnp.concatenate([src, jnp.zeros((pad,), jnp.int32)])
    dstp = jnp.concatenate([dst, pad_dst])
    idx = jnp.stack(
        [srcp.reshape(NW, CHUNKS, K), dstp.reshape(NW, CHUNKS, K)], axis=2
    )
    pad_dst2 = N + (jnp.arange(NW * PADCH * K, dtype=jnp.int32) % spare)
    padi = jnp.stack(
        [jnp.zeros((NW, PADCH, K), jnp.int32),
         pad_dst2.reshape(NW, PADCH, K)], axis=2
    )
    idxall = jnp.concatenate([idx, padi], axis=1)

    zeros_deg = jnp.zeros((ROWS_PER_TILE, DEGW), jnp.float32)
    ones_deg = jnp.ones((K, DEGW), jnp.float32)
    zeros_acc = jnp.zeros((ROWS_PER_TILE, D), jnp.float32)

    degp = _deg_kernel(dstp, ones_deg, zeros_deg)
    h, dinv = _tc_first(degp, x, W0)

    bs = [b0, b1, b2, b3]
    Ws = [W1, W2, W3, W4]
    for i in range(4):
        agg = _agg_kernel(h, idxall, zeros_acc)
        h = _tc_mid(agg, h, dinv, bs[i].reshape(1, D), Ws[i])
    agg = _agg_kernel(h, idxall, zeros_acc)
    return _tc_last(agg, h, dinv, b4.reshape(1, D))


# speedup vs baseline: 4.4895x; 4.4459x over previous
import functools

import jax
import jax.numpy as jnp
from jax import lax
from jax.experimental import pallas as pl
from jax.experimental.pallas import tpu as pltpu
from jax.experimental.pallas import tpu_sc as plsc

N = 10000
D = 128
E = 320000

NC = 2
NS = 16
NW = NC * NS
NPAD = 10240
ROWS_PER_TILE = NPAD // NS
EW = 10240
EPAD = NW * EW
K = 80
CHUNKS = EW // K
NBUF = 4
PADCH = NBUF
DEGW = 128

_BLK = 1000
_GRID = N // _BLK

_sc_mesh = plsc.VectorSubcoreMesh(
    core_axis_name="c", subcore_axis_name="s", num_cores=NC, num_subcores=NS
)


@functools.partial(
    pl.kernel,
    out_type=jax.ShapeDtypeStruct((NC, NPAD, DEGW), jnp.float32),
    mesh=_sc_mesh,
    scratch_types=[
        pltpu.VMEM_SHARED((NPAD, DEGW), jnp.float32),
        pltpu.VMEM((K,), jnp.int32),
        pltpu.VMEM((K, DEGW), jnp.float32),
    ],
)
def _deg_kernel(dst_hbm, ones_hbm, zeros_hbm, out_hbm, deg_sh, idx_v, ones_v):
    c = lax.axis_index("c")
    s = lax.axis_index("s")
    base = (s * NC + c) * EW
    soff = s * ROWS_PER_TILE
    pltpu.sync_copy(zeros_hbm, deg_sh.at[pl.ds(soff, ROWS_PER_TILE)])
    pltpu.sync_copy(ones_hbm, ones_v)
    plsc.subcore_barrier()

    def body(j, carry):
        pltpu.sync_copy(dst_hbm.at[pl.ds(base + j * K, K)], idx_v)
        pltpu.sync_copy(ones_v, deg_sh.at[idx_v], add=True)
        return carry

    lax.fori_loop(0, CHUNKS, body, 0)
    plsc.subcore_barrier()
    pltpu.sync_copy(
        deg_sh.at[pl.ds(soff, ROWS_PER_TILE)],
        out_hbm.at[c, pl.ds(soff, ROWS_PER_TILE)],
    )


@functools.partial(
    pl.kernel,
    out_type=jax.ShapeDtypeStruct((NC, NPAD, D), jnp.float32),
    mesh=_sc_mesh,
    scratch_types=[
        pltpu.VMEM_SHARED((NPAD, D), jnp.float32),
        [pltpu.VMEM((2, K), jnp.int32) for _ in range(NBUF)],
        [pltpu.VMEM((K, D), jnp.float32) for _ in range(NBUF)],
        [pltpu.SemaphoreType.DMA for _ in range(NBUF)],
        [pltpu.SemaphoreType.DMA for _ in range(NBUF)],
    ],
)
def _agg_kernel(hp_hbm, idx_hbm, zeros_hbm, out_hbm,
                acc_sh, idxb, rowsb, isem, gsem):
    c = lax.axis_index("c")
    s = lax.axis_index("s")
    wid = s * NC + c
    soff = s * ROWS_PER_TILE
    pltpu.sync_copy(zeros_hbm, acc_sh.at[pl.ds(soff, ROWS_PER_TILE)])

    for b in range(NBUF):
        pltpu.async_copy(idx_hbm.at[wid, b], idxb[b], isem[b])
    for b in range(2):
        pltpu.make_async_copy(idx_hbm.at[wid, b], idxb[b], isem[b]).wait()
        pltpu.async_copy(hp_hbm.at[idxb[b].at[0]], rowsb[b], gsem[b])
    plsc.subcore_barrier()

    def quad(p, carry):
        j0 = p * NBUF
        for i in range(NBUF):
            j = j0 + i
            cur = i
            nx2 = (i + 2) % NBUF
            pltpu.make_async_copy(idx_hbm.at[wid, j + 2], idxb[nx2],
                                  isem[nx2]).wait()
            pltpu.async_copy(hp_hbm.at[idxb[nx2].at[0]], rowsb[nx2],
                             gsem[nx2])
            pltpu.make_async_copy(hp_hbm.at[idxb[cur].at[0]], rowsb[cur],
                                  gsem[cur]).wait()
            pltpu.sync_copy(rowsb[cur], acc_sh.at[idxb[cur].at[1]], add=True)
            pltpu.async_copy(idx_hbm.at[wid, j + NBUF], idxb[cur], isem[cur])
        return carry

    lax.fori_loop(0, CHUNKS // NBUF, quad, 0)

    for b in range(2):
        pltpu.make_async_copy(hp_hbm.at[idxb[b].at[0]], rowsb[b],
                              gsem[b]).wait()
    for b in range(2, NBUF):
        pltpu.make_async_copy(idx_hbm.at[wid, b], idxb[b], isem[b]).wait()

    plsc.subcore_barrier()
    pltpu.sync_copy(
        acc_sh.at[pl.ds(soff, ROWS_PER_TILE)],
        out_hbm.at[c, pl.ds(soff, ROWS_PER_TILE)],
    )


def _dot(a, b):
    return lax.dot_general(
        a, b, (((1,), (0,)), ((), ())),
        precision=lax.Precision.HIGHEST,
        preferred_element_type=jnp.float32,
    )


def _tc_first_body(degp_ref, x_ref, w_ref, h_ref, dinv_ref):
    dp = degp_ref[...]
    deg = dp[0, :, 0:1] + dp[1, :, 0:1] + 1.0
    dinv = lax.rsqrt(deg)
    dinv_ref[...] = dinv
    h_ref[...] = _dot(x_ref[...] * dinv, w_ref[...])


def _tc_mid_body(agg_ref, hp_ref, dinv_ref, b_ref, w_ref, out_ref):
    a = agg_ref[...]
    dinv = dinv_ref[...]
    o = (a[0] + a[1] + hp_ref[...]) * dinv + b_ref[...]
    out_ref[...] = _dot(jnp.maximum(o, 0.0) * dinv, w_ref[...])


def _tc_last_body(agg_ref, hp_ref, dinv_ref, b_ref, out_ref):
    a = agg_ref[...]
    out_ref[...] = (a[0] + a[1] + hp_ref[...]) * dinv_ref[...] + b_ref[...]


_spec_agg = pl.BlockSpec((2, _BLK, D), lambda i: (0, i, 0))
_spec_deg = pl.BlockSpec((2, _BLK, DEGW), lambda i: (0, i, 0))
_spec_row = pl.BlockSpec((_BLK, D), lambda i: (i, 0))
_spec_col = pl.BlockSpec((_BLK, 1), lambda i: (i, 0))
_spec_b = pl.BlockSpec((1, D), lambda i: (0, 0))
_spec_w = pl.BlockSpec((D, D), lambda i: (0, 0))

_tc_first = pl.pallas_call(
    _tc_first_body,
    grid=(_GRID,),
    in_specs=[_spec_deg, _spec_row, _spec_w],
    out_specs=[_spec_row, _spec_col],
    out_shape=[
        jax.ShapeDtypeStruct((N, D), jnp.float32),
        jax.ShapeDtypeStruct((N, 1), jnp.float32),
    ],
)

_tc_mid = pl.pallas_call(
    _tc_mid_body,
    grid=(_GRID,),
    in_specs=[_spec_agg, _spec_row, _spec_col, _spec_b, _spec_w],
    out_specs=_spec_row,
    out_shape=jax.ShapeDtypeStruct((N, D), jnp.float32),
)

_tc_last = pl.pallas_call(
    _tc_last_body,
    grid=(_GRID,),
    in_specs=[_spec_agg, _spec_row, _spec_col, _spec_b],
    out_specs=_spec_row,
    out_shape=jax.ShapeDtypeStruct((N, D), jnp.float32),
)


def kernel(x, edge_index, W0, b0, W1, b1, W2, b2, W3, b3, W4, b4):
    src = edge_index[0]
    dst = edge_index[1]
    pad = EPAD - E
    spare = NPAD - N
    pad_dst = N + (jnp.arange(pad, dtype=jnp.int32) % spare)
    pad_src = jnp.arange(pad, dtype=jnp.int32) % N
    srcp = jnp.concatenate([src, pad_src])
    dstp = jnp.concatenate([dst, pad_dst])
    idx = jnp.stack(
        [srcp.reshape(NW, CHUNKS, K), dstp.reshape(NW, CHUNKS, K)], axis=2
    )
    pad_dst2 = N + (jnp.arange(NW * PADCH * K, dtype=jnp.int32) % spare)
    pad_src2 = jnp.arange(NW * PADCH * K, dtype=jnp.int32) % N
    padi = jnp.stack(
        [pad_src2.reshape(NW, PADCH, K),
         pad_dst2.reshape(NW, PADCH, K)], axis=2
    )
    idxall = jnp.concatenate([idx, padi], axis=1)

    zeros_deg = jnp.zeros((ROWS_PER_TILE, DEGW), jnp.float32)
    ones_deg = jnp.ones((K, DEGW), jnp.float32)
    zeros_acc = jnp.zeros((ROWS_PER_TILE, D), jnp.float32)

    degp = _deg_kernel(dstp, ones_deg, zeros_deg)
    h, dinv = _tc_first(degp, x, W0)

    bs = [b0, b1, b2, b3]
    Ws = [W1, W2, W3, W4]
    for i in range(4):
        agg = _agg_kernel(h, idxall, zeros_acc)
        h = _tc_mid(agg, h, dinv, bs[i].reshape(1, D), Ws[i])
    agg = _agg_kernel(h, idxall, zeros_acc)
    return _tc_last(agg, h, dinv, b4.reshape(1, D))


# speedup vs baseline: 4.5158x; 1.0059x over previous
import functools

import jax
import jax.numpy as jnp
from jax import lax
from jax.experimental import pallas as pl
from jax.experimental.pallas import tpu as pltpu
from jax.experimental.pallas import tpu_sc as plsc

N = 10000
D = 128
E = 320000

NC = 2
NS = 16
NW = NC * NS
NPAD = 10240
ROWS_PER_TILE = NPAD // NS
EW = 10240
EPAD = NW * EW
K = 80
CHUNKS = EW // K
NBUF = 4
PADCH = NBUF
DEGW = 128

_BLK = 1000
_GRID = N // _BLK

_sc_mesh = plsc.VectorSubcoreMesh(
    core_axis_name="c", subcore_axis_name="s", num_cores=NC, num_subcores=NS
)


@functools.partial(
    pl.kernel,
    out_type=jax.ShapeDtypeStruct((NC, NPAD, DEGW), jnp.float32),
    mesh=_sc_mesh,
    scratch_types=[
        pltpu.VMEM_SHARED((NPAD, DEGW), jnp.float32),
        pltpu.VMEM((K,), jnp.int32),
        pltpu.VMEM((K, DEGW), jnp.float32),
    ],
)
def _deg_kernel(dst_hbm, ones_hbm, zeros_hbm, out_hbm, deg_sh, idx_v, ones_v):
    c = lax.axis_index("c")
    s = lax.axis_index("s")
    base = (s * NC + c) * EW
    soff = s * ROWS_PER_TILE
    pltpu.sync_copy(zeros_hbm, deg_sh.at[pl.ds(soff, ROWS_PER_TILE)])
    pltpu.sync_copy(ones_hbm, ones_v)
    plsc.subcore_barrier()

    def body(j, carry):
        pltpu.sync_copy(dst_hbm.at[pl.ds(base + j * K, K)], idx_v)
        pltpu.sync_copy(ones_v, deg_sh.at[idx_v], add=True)
        return carry

    lax.fori_loop(0, CHUNKS, body, 0)
    plsc.subcore_barrier()
    pltpu.sync_copy(
        deg_sh.at[pl.ds(soff, ROWS_PER_TILE)],
        out_hbm.at[c, pl.ds(soff, ROWS_PER_TILE)],
    )


@functools.partial(
    pl.kernel,
    out_type=jax.ShapeDtypeStruct((NC, NPAD, D), jnp.float32),
    mesh=_sc_mesh,
    scratch_types=[
        pltpu.VMEM_SHARED((NPAD, D), jnp.float32),
        [pltpu.VMEM((2, K), jnp.int32) for _ in range(NBUF)],
        [pltpu.VMEM((K, D), jnp.float32) for _ in range(NBUF)],
        [pltpu.SemaphoreType.DMA for _ in range(NBUF)],
        [pltpu.SemaphoreType.DMA for _ in range(NBUF)],
    ],
)
def _agg_kernel(hp_hbm, idx_hbm, zeros_hbm, out_hbm,
                acc_sh, idxb, rowsb, isem, gsem):
    c = lax.axis_index("c")
    s = lax.axis_index("s")
    wid = s * NC + c
    soff = s * ROWS_PER_TILE
    pltpu.sync_copy(zeros_hbm, acc_sh.at[pl.ds(soff, ROWS_PER_TILE)])

    for b in range(NBUF):
        pltpu.async_copy(idx_hbm.at[wid, b], idxb[b], isem[b])
    for b in range(2):
        pltpu.make_async_copy(idx_hbm.at[wid, b], idxb[b], isem[b]).wait()
        pltpu.async_copy(hp_hbm.at[idxb[b].at[0]], rowsb[b], gsem[b])
    plsc.subcore_barrier()

    def quad(p, carry):
        j0 = p * NBUF
        for i in range(NBUF):
            j = j0 + i
            cur = i
            nx2 = (i + 2) % NBUF
            pltpu.make_async_copy(idx_hbm.at[wid, j + 2], idxb[nx2],
                                  isem[nx2]).wait()
            pltpu.async_copy(hp_hbm.at[idxb[nx2].at[0]], rowsb[nx2],
                             gsem[nx2])
            pltpu.make_async_copy(hp_hbm.at[idxb[cur].at[0]], rowsb[cur],
                                  gsem[cur]).wait()
            pltpu.sync_copy(rowsb[cur], acc_sh.at[idxb[cur].at[1]], add=True)
            pltpu.async_copy(idx_hbm.at[wid, j + NBUF], idxb[cur], isem[cur])
        return carry

    lax.fori_loop(0, CHUNKS // NBUF, quad, 0)

    for b in range(2):
        pltpu.make_async_copy(hp_hbm.at[idxb[b].at[0]], rowsb[b],
                              gsem[b]).wait()
    for b in range(2, NBUF):
        pltpu.make_async_copy(idx_hbm.at[wid, b], idxb[b], isem[b]).wait()

    plsc.subcore_barrier()
    pltpu.sync_copy(
        acc_sh.at[pl.ds(soff, ROWS_PER_TILE)],
        out_hbm.at[c, pl.ds(soff, ROWS_PER_TILE)],
    )


def _dot(a, b):
    return lax.dot_general(
        a, b, (((1,), (0,)), ((), ())),
        precision=lax.Precision.HIGHEST,
        preferred_element_type=jnp.float32,
    )


def _tc_mm0_body(x_ref, w_ref, h_ref):
    h_ref[...] = _dot(x_ref[...], w_ref[...])


def _tc_scale_body(degp_ref, h0_ref, h_ref, dinv_ref):
    dp = degp_ref[...]
    deg = dp[0, :, 0:1] + dp[1, :, 0:1] + 1.0
    dinv = lax.rsqrt(deg)
    dinv_ref[...] = dinv
    h_ref[...] = h0_ref[...] * dinv


def _tc_mid_body(agg_ref, hp_ref, dinv_ref, b_ref, w_ref, out_ref):
    a = agg_ref[...]
    dinv = dinv_ref[...]
    o = (a[0] + a[1] + hp_ref[...]) * dinv + b_ref[...]
    out_ref[...] = _dot(jnp.maximum(o, 0.0) * dinv, w_ref[...])


def _tc_last_body(agg_ref, hp_ref, dinv_ref, b_ref, out_ref):
    a = agg_ref[...]
    out_ref[...] = (a[0] + a[1] + hp_ref[...]) * dinv_ref[...] + b_ref[...]


_spec_agg = pl.BlockSpec((2, _BLK, D), lambda i: (0, i, 0))
_spec_deg = pl.BlockSpec((2, _BLK, DEGW), lambda i: (0, i, 0))
_spec_row = pl.BlockSpec((_BLK, D), lambda i: (i, 0))
_spec_col = pl.BlockSpec((_BLK, 1), lambda i: (i, 0))
_spec_b = pl.BlockSpec((1, D), lambda i: (0, 0))
_spec_w = pl.BlockSpec((D, D), lambda i: (0, 0))

_tc_mm0 = pl.pallas_call(
    _tc_mm0_body,
    grid=(_GRID,),
    in_specs=[_spec_row, _spec_w],
    out_specs=_spec_row,
    out_shape=jax.ShapeDtypeStruct((N, D), jnp.float32),
)

_tc_scale = pl.pallas_call(
    _tc_scale_body,
    grid=(_GRID,),
    in_specs=[_spec_deg, _spec_row],
    out_specs=[_spec_row, _spec_col],
    out_shape=[
        jax.ShapeDtypeStruct((N, D), jnp.float32),
        jax.ShapeDtypeStruct((N, 1), jnp.float32),
    ],
)

_tc_mid = pl.pallas_call(
    _tc_mid_body,
    grid=(_GRID,),
    in_specs=[_spec_agg, _spec_row, _spec_col, _spec_b, _spec_w],
    out_specs=_spec_row,
    out_shape=jax.ShapeDtypeStruct((N, D), jnp.float32),
)

_tc_last = pl.pallas_call(
    _tc_last_body,
    grid=(_GRID,),
    in_specs=[_spec_agg, _spec_row, _spec_col, _spec_b],
    out_specs=_spec_row,
    out_shape=jax.ShapeDtypeStruct((N, D), jnp.float32),
)


def kernel(x, edge_index, W0, b0, W1, b1, W2, b2, W3, b3, W4, b4):
    src = edge_index[0]
    dst = edge_index[1]
    pad = EPAD - E
    spare = NPAD - N
    pad_dst = N + (jnp.arange(pad, dtype=jnp.int32) % spare)
    pad_src = jnp.arange(pad, dtype=jnp.int32) % N
    srcp = jnp.concatenate([src, pad_src])
    dstp = jnp.concatenate([dst, pad_dst])
    idx = jnp.stack(
        [srcp.reshape(NW, CHUNKS, K), dstp.reshape(NW, CHUNKS, K)], axis=2
    )
    pad_dst2 = N + (jnp.arange(NW * PADCH * K, dtype=jnp.int32) % spare)
    pad_src2 = jnp.arange(NW * PADCH * K, dtype=jnp.int32) % N
    padi = jnp.stack(
        [pad_src2.reshape(NW, PADCH, K),
         pad_dst2.reshape(NW, PADCH, K)], axis=2
    )
    idxall = jnp.concatenate([idx, padi], axis=1)

    zeros_deg = jnp.zeros((ROWS_PER_TILE, DEGW), jnp.float32)
    ones_deg = jnp.ones((K, DEGW), jnp.float32)
    zeros_acc = jnp.zeros((ROWS_PER_TILE, D), jnp.float32)

    degp = _deg_kernel(dstp, ones_deg, zeros_deg)
    h0 = _tc_mm0(x, W0)
    h, dinv = _tc_scale(degp, h0)

    bs = [b0, b1, b2, b3]
    Ws = [W1, W2, W3, W4]
    for i in range(4):
        agg = _agg_kernel(h, idxall, zeros_acc)
        h = _tc_mid(agg, h, dinv, bs[i].reshape(1, D), Ws[i])
    agg = _agg_kernel(h, idxall, zeros_acc)
    return _tc_last(agg, h, dinv, b4.reshape(1, D))
